# baseline (speedup 1.0000x reference)
import jax
import jax.numpy as jnp
from jax import lax
from jax.experimental import pallas as pl
from jax.experimental.pallas import tpu as pltpu
from jax.experimental.pallas import tpu_sc as plsc

_N = 10000
_E = 320000
_D = 128
_DE = 16
_G = 64
_TS = 512
_TASKS = 10

_NC = 2
_NS = 16
_NW = _NC * _NS
_EPT = _E // _NW
_CH = 112
_NFULL = _EPT // _CH
_TAIL = _EPT - _NFULL * _CH
_RPT = 624
_RTAIL = _N - _NS * _RPT
_NCHUNK = _N // _CH
_NTAIL = _N - _NCHUNK * _CH
_TSPT = _TS // _NS


def _sc_mesh():
    return plsc.VectorSubcoreMesh(
        core_axis_name="c", subcore_axis_name="s",
        num_cores=_NC, num_subcores=_NS)


def _zero_acc(acc_sh, z128, s):
    row0 = pl.multiple_of(s * _RPT, 8)
    pltpu.sync_copy(z128.at[pl.ds(0, _RPT)], acc_sh.at[pl.ds(row0, _RPT)])

    @pl.when(s == _NS - 1)
    def _():
        pltpu.sync_copy(z128.at[pl.ds(0, _RTAIL)],
                        acc_sh.at[pl.ds(_NS * _RPT, _RTAIL)])
    return row0


def _copy_out(acc_sh, out_hbm, c, s, row0):
    pltpu.sync_copy(acc_sh.at[pl.ds(row0, _RPT)],
                    out_hbm.at[c, pl.ds(row0, _RPT)])

    @pl.when(s == _NS - 1)
    def _():
        pltpu.sync_copy(acc_sh.at[pl.ds(_NS * _RPT, _RTAIL)],
                        out_hbm.at[c, pl.ds(_NS * _RPT, _RTAIL)])


_RING = 3
_NRND = _NFULL // _RING
_NLEFT = _NFULL - _NRND * _RING


def _agg_body(ei_hbm, h_hbm, z128, out_hbm,
              acc_sh, s0, s1, s2, d0, d1, d2, r0, r1, r2,
              src_t, dst_t, rows_t, ga, gb, gc, sa, sb, sc, sem_x):
    srcs = [s0, s1, s2]
    dsts = [d0, d1, d2]
    rows = [r0, r1, r2]
    gsems = [ga, gb, gc]
    ssems = [sa, sb, sc]
    c = lax.axis_index("c")
    s = lax.axis_index("s")
    wid = c * _NS + s
    row0 = _zero_acc(acc_sh, z128, s)
    base = wid * _EPT
    plsc.subcore_barrier()

    def arm(i, b):
        pltpu.sync_copy(ei_hbm.at[pl.ds(base + i * _CH, _CH)], srcs[b])
        pltpu.sync_copy(ei_hbm.at[pl.ds(_E + base + i * _CH, _CH)], dsts[b])
        pltpu.async_copy(h_hbm.at[srcs[b]], rows[b], gsems[b])

    for b in range(_RING):
        arm(b, b)

    def body(r, carry):
        i0 = r * _RING
        for b in range(_RING):
            pltpu.make_async_copy(h_hbm.at[srcs[b]], rows[b],
                                  gsems[b]).wait()
            pltpu.async_copy(rows[b], acc_sh.at[dsts[b]], ssems[b],
                             add=True)
        for b in range(_RING):
            ni = i0 + _RING + b
            pltpu.make_async_copy(rows[b], acc_sh.at[dsts[b]],
                                  ssems[b]).wait()

            @pl.when(ni < _NFULL)
            def _():
                arm(ni, b)
        return carry

    lax.fori_loop(0, _NRND, body, 0)

    for b in range(_NLEFT):
        pltpu.make_async_copy(h_hbm.at[srcs[b]], rows[b], gsems[b]).wait()
        pltpu.async_copy(rows[b], acc_sh.at[dsts[b]], ssems[b], add=True)
    for b in range(_NLEFT):
        pltpu.make_async_copy(rows[b], acc_sh.at[dsts[b]], ssems[b]).wait()

    off = base + _NFULL * _CH
    pltpu.sync_copy(ei_hbm.at[pl.ds(off, _TAIL)], src_t)
    pltpu.sync_copy(ei_hbm.at[pl.ds(_E + off, _TAIL)], dst_t)
    pltpu.async_copy(h_hbm.at[src_t], rows_t, sem_x).wait()
    pltpu.sync_copy(rows_t, acc_sh.at[dst_t], add=True)

    plsc.subcore_barrier()
    _copy_out(acc_sh, out_hbm, c, s, row0)


def _regpad(flat, wide, nrows):
    def rp(p, carry):
        for q in range(8):
            r = p * 8 + q
            wide[r, pl.ds(0, _DE)] = flat[pl.ds(r * _DE, _DE)]
        return carry
    lax.fori_loop(0, nrows // 8, rp, 0)


def _ea_body(ei_hbm, eaflat_hbm, z128, out_hbm,
             acc_sh, dst_a, dst_b, flat_a, flat_b, wide_a, wide_b,
             dst_t, flat_t, wide_t, sem_a, sem_b):
    c = lax.axis_index("c")
    s = lax.axis_index("s")
    wid = c * _NS + s
    row0 = _zero_acc(acc_sh, z128, s)
    pltpu.sync_copy(z128.at[pl.ds(0, _CH)], wide_a)
    pltpu.sync_copy(z128.at[pl.ds(0, _CH)], wide_b)
    pltpu.sync_copy(z128.at[pl.ds(0, _TAIL)], wide_t)
    base = wid * _EPT
    plsc.subcore_barrier()

    def rdsrc(i):
        return eaflat_hbm.at[pl.ds((base + i * _CH) * _DE, _CH * _DE)]

    pltpu.sync_copy(ei_hbm.at[pl.ds(_E + base, _CH)], dst_a)
    pltpu.async_copy(rdsrc(0), flat_a, sem_a)

    npair = _NFULL // 2

    def body(p, carry):
        i0 = p * 2
        i1 = i0 + 1
        pltpu.sync_copy(ei_hbm.at[pl.ds(_E + base + i1 * _CH, _CH)], dst_b)
        cd_b = pltpu.async_copy(rdsrc(i1), flat_b, sem_b)
        pltpu.make_async_copy(rdsrc(i0), flat_a, sem_a).wait()
        _regpad(flat_a, wide_a, _CH)
        pltpu.sync_copy(wide_a, acc_sh.at[dst_a], add=True)

        @pl.when(p < npair - 1)
        def _():
            pltpu.sync_copy(ei_hbm.at[pl.ds(_E + base + (i0 + 2) * _CH, _CH)],
                            dst_a)
            pltpu.async_copy(rdsrc(i0 + 2), flat_a, sem_a)

        cd_b.wait()
        _regpad(flat_b, wide_b, _CH)
        pltpu.sync_copy(wide_b, acc_sh.at[dst_b], add=True)
        return carry

    lax.fori_loop(0, npair, body, 0)

    if _NFULL % 2:
        i = _NFULL - 1
        pltpu.sync_copy(ei_hbm.at[pl.ds(_E + base + i * _CH, _CH)], dst_a)
        pltpu.async_copy(rdsrc(i), flat_a, sem_a).wait()
        _regpad(flat_a, wide_a, _CH)
        pltpu.sync_copy(wide_a, acc_sh.at[dst_a], add=True)

    off = base + _NFULL * _CH
    pltpu.sync_copy(ei_hbm.at[pl.ds(_E + off, _TAIL)], dst_t)
    pltpu.async_copy(eaflat_hbm.at[pl.ds(off * _DE, _TAIL * _DE)],
                     flat_t, sem_a).wait()
    _regpad(flat_t, wide_t, _TAIL)
    pltpu.sync_copy(wide_t, acc_sh.at[dst_t], add=True)

    plsc.subcore_barrier()
    _copy_out(acc_sh, out_hbm, c, s, row0)


def _pool_body(h_hbm, sid_hbm, z128, ones_hbm,
               sums_hbm, cnt_hbm,
               acc_sh, cnt_sh, rows_v, sid_v, rows_t, sid_t, ones_v):
    c = lax.axis_index("c")
    s = lax.axis_index("s")
    wid = c * _NS + s
    r0 = pl.multiple_of(s * _TSPT, 8)
    pltpu.sync_copy(z128.at[pl.ds(0, _TSPT)], acc_sh.at[pl.ds(r0, _TSPT)])
    pltpu.sync_copy(z128.at[pl.ds(0, _TSPT)], cnt_sh.at[pl.ds(r0, _TSPT)])
    pltpu.sync_copy(ones_hbm, ones_v)
    plsc.subcore_barrier()

    for j in range(3):
        k = wid + _NW * j

        @pl.when(k < _NCHUNK)
        def _():
            off = pl.multiple_of(k * _CH, 8)
            pltpu.sync_copy(h_hbm.at[pl.ds(off, _CH)], rows_v)
            pltpu.sync_copy(sid_hbm.at[pl.ds(off, _CH)], sid_v)
            pltpu.sync_copy(rows_v, acc_sh.at[sid_v], add=True)
            pltpu.sync_copy(ones_v, cnt_sh.at[sid_v], add=True)

    @pl.when(wid == _NW - 1)
    def _():
        off = _NCHUNK * _CH
        pltpu.sync_copy(h_hbm.at[pl.ds(off, _NTAIL)], rows_t)
        pltpu.sync_copy(sid_hbm.at[pl.ds(off, _NTAIL)], sid_t)
        pltpu.sync_copy(rows_t, acc_sh.at[sid_t], add=True)
        pltpu.sync_copy(ones_v.at[pl.ds(0, _NTAIL)], cnt_sh.at[sid_t], add=True)

    plsc.subcore_barrier()
    pltpu.sync_copy(acc_sh.at[pl.ds(r0, _TSPT)], sums_hbm.at[c, pl.ds(r0, _TSPT)])
    pltpu.sync_copy(cnt_sh.at[pl.ds(r0, _TSPT)], cnt_hbm.at[c, pl.ds(r0, _TSPT)])


def _dense_body(h_ref, parts_ref, ea_parts_ref,
                wr_ref, wn_ref, wep_ref, b_ref, g_ref, be_ref, out_ref):
    h = h_ref[...]
    nbr = parts_ref[0] + parts_ref[1]
    ea = ea_parts_ref[0] + ea_parts_ref[1]
    nbr = nbr + jnp.dot(ea, wep_ref[...], preferred_element_type=jnp.float32)
    z = (jnp.dot(h, wr_ref[...], preferred_element_type=jnp.float32)
         + jnp.dot(nbr, wn_ref[...], preferred_element_type=jnp.float32)
         + b_ref[...])
    m = jnp.mean(z, axis=0, keepdims=True)
    d = z - m
    v = jnp.mean(d * d, axis=0, keepdims=True)
    out_ref[...] = jnp.maximum(d * jax.lax.rsqrt(v + 1e-5) * g_ref[...]
                               + be_ref[...], 0.0)


def _head_body(sums_ref, cnt_ref, wf1_ref, bf1_ref, wf2_ref, bf2_ref, out_ref):
    sums = sums_ref[0] + sums_ref[1]
    cnt = cnt_ref[0] + cnt_ref[1]
    csafe = jnp.maximum(cnt[:, 0:1], 1.0)
    hsub = sums / csafe
    rows = lax.broadcasted_iota(jnp.int32, (_G, _TS), 0)
    cols = lax.broadcasted_iota(jnp.int32, (_G, _TS), 1)
    pool = jnp.where(cols // (_TS // _G) == rows, 1.0 / (_TS // _G), 0.0)
    hg = jnp.dot(pool, hsub, preferred_element_type=jnp.float32)
    t = jnp.maximum(jnp.dot(hg, wf1_ref[...], preferred_element_type=jnp.float32)
                    + bf1_ref[...], 0.0)
    out_ref[...] = (jnp.dot(t, wf2_ref[...], preferred_element_type=jnp.float32)
                    + bf2_ref[...])


def _dense(h, parts, ea_parts, wr, wn, wep, b, g, be):
    return pl.pallas_call(
        _dense_body,
        out_shape=jax.ShapeDtypeStruct((_N, _D), jnp.float32),
    )(h, parts, ea_parts, wr, wn, wep,
      b.reshape(1, -1), g.reshape(1, -1), be.reshape(1, -1))


def _make_agg():
    f32 = jnp.float32
    i32 = jnp.int32
    return pl.kernel(
        _agg_body,
        out_type=jax.ShapeDtypeStruct((_NC, _N, _D), f32),
        mesh=_sc_mesh(),
        scratch_types=(
            [pltpu.VMEM_SHARED((_N, _D), f32)]
            + [pltpu.VMEM((_CH,), i32)] * 6
            + [pltpu.VMEM((_CH, _D), f32)] * 3
            + [pltpu.VMEM((_TAIL,), i32)] * 2
            + [pltpu.VMEM((_TAIL, _D), f32)]
            + [pltpu.SemaphoreType.DMA] * 7
        ))


def kernel(x, edge_index, edge_attr, batch, subgraph_batch, num_subgraphs,
           subgraph_id_batch, W_root0, W_nbr0, We0, b0, g0, be0,
           W_root1, W_nbr1, We1, b1, g1, be1, Wf1, bf1, Wf2, bf2):
    f32 = jnp.float32
    i32 = jnp.int32
    z128 = jnp.zeros((_RPT, _D), f32)
    ones = jnp.ones((_CH, _D), f32)
    ei_flat = edge_index.reshape(-1)
    We0p = jnp.pad(We0, ((0, _D - _DE), (0, 0)))
    We1p = jnp.pad(We1, ((0, _D - _DE), (0, 0)))
    ea_flat = edge_attr.reshape(-1)

    agg = _make_agg()
    parts0 = agg(ei_flat, x, z128)

    ea_k = pl.kernel(
        _ea_body,
        out_type=jax.ShapeDtypeStruct((_NC, _N, _D), f32),
        mesh=_sc_mesh(),
        scratch_types=[
            pltpu.VMEM_SHARED((_N, _D), f32),
            pltpu.VMEM((_CH,), i32), pltpu.VMEM((_CH,), i32),
            pltpu.VMEM((_CH * _DE,), f32), pltpu.VMEM((_CH * _DE,), f32),
            pltpu.VMEM((_CH, _D), f32), pltpu.VMEM((_CH, _D), f32),
            pltpu.VMEM((_TAIL,), i32), pltpu.VMEM((_TAIL * _DE,), f32),
            pltpu.VMEM((_TAIL, _D), f32),
            pltpu.SemaphoreType.DMA, pltpu.SemaphoreType.DMA,
        ])
    ea_parts = ea_k(ei_flat, ea_flat, z128)

    h1 = _dense(x, parts0, ea_parts, W_root0, W_nbr0, We0p, b0, g0, be0)
    parts1 = _make_agg()(ei_flat, h1, z128)
    h2 = _dense(h1, parts1, ea_parts, W_root1, W_nbr1, We1p, b1, g1, be1)

    tmp = jnp.concatenate([jnp.zeros((1,), i32),
                           jnp.cumsum(num_subgraphs, dtype=i32)])
    sid = subgraph_batch + tmp[batch]

    pool = pl.kernel(
        _pool_body,
        out_type=(jax.ShapeDtypeStruct((_NC, _TS, _D), f32),
                  jax.ShapeDtypeStruct((_NC, _TS, _D), f32)),
        mesh=_sc_mesh(),
        scratch_types=[
            pltpu.VMEM_SHARED((_TS, _D), f32),
            pltpu.VMEM_SHARED((_TS, _D), f32),
            pltpu.VMEM((_CH, _D), f32), pltpu.VMEM((_CH,), i32),
            pltpu.VMEM((_NTAIL, _D), f32), pltpu.VMEM((_NTAIL,), i32),
            pltpu.VMEM((_CH, _D), f32),
        ])
    sums, cnts = pool(h2, sid, z128, ones)

    out = pl.pallas_call(
        _head_body,
        out_shape=jax.ShapeDtypeStruct((_G, _TASKS), jnp.float32),
    )(sums, cnts, Wf1, bf1.reshape(1, -1), Wf2, bf2.reshape(1, -1))
    return out


# speedup vs baseline: 1.1092x; 1.1092x over previous
import jax
import jax.numpy as jnp
from jax import lax
from jax.experimental import pallas as pl
from jax.experimental.pallas import tpu as pltpu
from jax.experimental.pallas import tpu_sc as plsc

_N = 10000
_E = 320000
_D = 128
_DE = 16
_G = 64
_TS = 512
_TASKS = 10

_NC = 2
_NS = 16
_NW = _NC * _NS
_EPT = _E // _NW
_CH = 128
_NFULL = _EPT // _CH
_TAIL = _EPT - _NFULL * _CH
_RPT = 624
_RTAIL = _N - _NS * _RPT
_NCHUNK = _N // _CH
_NTAIL = _N - _NCHUNK * _CH
_TSPT = _TS // _NS


def _sc_mesh():
    return plsc.VectorSubcoreMesh(
        core_axis_name="c", subcore_axis_name="s",
        num_cores=_NC, num_subcores=_NS)


def _zero_acc(acc_sh, z128, s):
    row0 = pl.multiple_of(s * _RPT, 8)
    pltpu.sync_copy(z128.at[pl.ds(0, _RPT)], acc_sh.at[pl.ds(row0, _RPT)])

    @pl.when(s == _NS - 1)
    def _():
        pltpu.sync_copy(z128.at[pl.ds(0, _RTAIL)],
                        acc_sh.at[pl.ds(_NS * _RPT, _RTAIL)])
    return row0


def _copy_out(acc_sh, out_hbm, c, s, row0):
    pltpu.sync_copy(acc_sh.at[pl.ds(row0, _RPT)],
                    out_hbm.at[c, pl.ds(row0, _RPT)])

    @pl.when(s == _NS - 1)
    def _():
        pltpu.sync_copy(acc_sh.at[pl.ds(_NS * _RPT, _RTAIL)],
                        out_hbm.at[c, pl.ds(_NS * _RPT, _RTAIL)])


def _agg_body(ei_hbm, h_hbm, z128, out_hbm,
              acc_sh, srcall, dst_a, dst_b, rows_a, rows_b,
              dst_t, rows_t, sem_a, sem_b):
    c = lax.axis_index("c")
    s = lax.axis_index("s")
    wid = c * _NS + s
    row0 = _zero_acc(acc_sh, z128, s)
    base = wid * _EPT
    pltpu.sync_copy(ei_hbm.at[pl.ds(base, _EPT)], srcall)
    plsc.subcore_barrier()

    def gsrc(i):
        return h_hbm.at[srcall.at[pl.ds(i * _CH, _CH)]]

    pltpu.sync_copy(ei_hbm.at[pl.ds(_E + base, _CH)], dst_a)
    pltpu.async_copy(gsrc(0), rows_a, sem_a)

    npair = _NFULL // 2

    def body(p, carry):
        i0 = p * 2
        i1 = i0 + 1
        pltpu.sync_copy(ei_hbm.at[pl.ds(_E + base + i1 * _CH, _CH)], dst_b)
        cd_b = pltpu.async_copy(gsrc(i1), rows_b, sem_b)
        pltpu.make_async_copy(gsrc(i0), rows_a, sem_a).wait()
        pltpu.sync_copy(rows_a, acc_sh.at[dst_a], add=True)

        @pl.when(p < npair - 1)
        def _():
            pltpu.sync_copy(ei_hbm.at[pl.ds(_E + base + (i0 + 2) * _CH, _CH)],
                            dst_a)
            pltpu.async_copy(gsrc(i0 + 2), rows_a, sem_a)

        cd_b.wait()
        pltpu.sync_copy(rows_b, acc_sh.at[dst_b], add=True)
        return carry

    lax.fori_loop(0, npair, body, 0)

    off = base + _NFULL * _CH
    pltpu.sync_copy(ei_hbm.at[pl.ds(_E + off, _TAIL)], dst_t)
    pltpu.async_copy(
        h_hbm.at[srcall.at[pl.ds(_NFULL * _CH, _TAIL)]], rows_t, sem_a).wait()
    pltpu.sync_copy(rows_t, acc_sh.at[dst_t], add=True)

    plsc.subcore_barrier()
    _copy_out(acc_sh, out_hbm, c, s, row0)


def _regpad(flat, wide, nrows):
    def rp(p, carry):
        for q in range(8):
            r = p * 8 + q
            wide[r, pl.ds(0, _DE)] = flat[pl.ds(r * _DE, _DE)]
        return carry
    lax.fori_loop(0, nrows // 8, rp, 0)


def _ea_body(ei_hbm, eaflat_hbm, z128, out_hbm,
             acc_sh, dst_a, dst_b, flat_a, flat_b, wide_a, wide_b,
             dst_t, flat_t, wide_t, sem_a, sem_b):
    c = lax.axis_index("c")
    s = lax.axis_index("s")
    wid = c * _NS + s
    row0 = _zero_acc(acc_sh, z128, s)
    pltpu.sync_copy(z128.at[pl.ds(0, _CH)], wide_a)
    pltpu.sync_copy(z128.at[pl.ds(0, _CH)], wide_b)
    pltpu.sync_copy(z128.at[pl.ds(0, _TAIL)], wide_t)
    base = wid * _EPT
    plsc.subcore_barrier()

    def rdsrc(i):
        return eaflat_hbm.at[pl.ds((base + i * _CH) * _DE, _CH * _DE)]

    pltpu.sync_copy(ei_hbm.at[pl.ds(_E + base, _CH)], dst_a)
    pltpu.async_copy(rdsrc(0), flat_a, sem_a)

    npair = _NFULL // 2

    def body(p, carry):
        i0 = p * 2
        i1 = i0 + 1
        pltpu.sync_copy(ei_hbm.at[pl.ds(_E + base + i1 * _CH, _CH)], dst_b)
        cd_b = pltpu.async_copy(rdsrc(i1), flat_b, sem_b)
        pltpu.make_async_copy(rdsrc(i0), flat_a, sem_a).wait()
        _regpad(flat_a, wide_a, _CH)
        pltpu.sync_copy(wide_a, acc_sh.at[dst_a], add=True)

        @pl.when(p < npair - 1)
        def _():
            pltpu.sync_copy(ei_hbm.at[pl.ds(_E + base + (i0 + 2) * _CH, _CH)],
                            dst_a)
            pltpu.async_copy(rdsrc(i0 + 2), flat_a, sem_a)

        cd_b.wait()
        _regpad(flat_b, wide_b, _CH)
        pltpu.sync_copy(wide_b, acc_sh.at[dst_b], add=True)
        return carry

    lax.fori_loop(0, npair, body, 0)

    off = base + _NFULL * _CH
    pltpu.sync_copy(ei_hbm.at[pl.ds(_E + off, _TAIL)], dst_t)
    pltpu.async_copy(eaflat_hbm.at[pl.ds(off * _DE, _TAIL * _DE)],
                     flat_t, sem_a).wait()
    _regpad(flat_t, wide_t, _TAIL)
    pltpu.sync_copy(wide_t, acc_sh.at[dst_t], add=True)

    plsc.subcore_barrier()
    _copy_out(acc_sh, out_hbm, c, s, row0)


def _pool_body(h_hbm, sid_hbm, z128, ones_hbm,
               sums_hbm, cnt_hbm,
               acc_sh, cnt_sh, rows_v, sid_v, rows_t, sid_t, ones_v):
    c = lax.axis_index("c")
    s = lax.axis_index("s")
    wid = c * _NS + s
    r0 = pl.multiple_of(s * _TSPT, 8)
    pltpu.sync_copy(z128.at[pl.ds(0, _TSPT)], acc_sh.at[pl.ds(r0, _TSPT)])
    pltpu.sync_copy(z128.at[pl.ds(0, _TSPT)], cnt_sh.at[pl.ds(r0, _TSPT)])
    pltpu.sync_copy(ones_hbm, ones_v)
    plsc.subcore_barrier()

    for j in range(3):
        k = wid + _NW * j

        @pl.when(k < _NCHUNK)
        def _():
            off = pl.multiple_of(k * _CH, 8)
            pltpu.sync_copy(h_hbm.at[pl.ds(off, _CH)], rows_v)
            pltpu.sync_copy(sid_hbm.at[pl.ds(off, _CH)], sid_v)
            pltpu.sync_copy(rows_v, acc_sh.at[sid_v], add=True)
            pltpu.sync_copy(ones_v, cnt_sh.at[sid_v], add=True)

    @pl.when(wid == _NW - 1)
    def _():
        off = _NCHUNK * _CH
        pltpu.sync_copy(h_hbm.at[pl.ds(off, _NTAIL)], rows_t)
        pltpu.sync_copy(sid_hbm.at[pl.ds(off, _NTAIL)], sid_t)
        pltpu.sync_copy(rows_t, acc_sh.at[sid_t], add=True)
        pltpu.sync_copy(ones_v.at[pl.ds(0, _NTAIL)], cnt_sh.at[sid_t], add=True)

    plsc.subcore_barrier()
    pltpu.sync_copy(acc_sh.at[pl.ds(r0, _TSPT)], sums_hbm.at[c, pl.ds(r0, _TSPT)])
    pltpu.sync_copy(cnt_sh.at[pl.ds(r0, _TSPT)], cnt_hbm.at[c, pl.ds(r0, _TSPT)])


def _dense_body(h_ref, parts_ref, ea_parts_ref,
                wr_ref, wn_ref, wep_ref, b_ref, g_ref, be_ref, out_ref):
    h = h_ref[...]
    nbr = parts_ref[0] + parts_ref[1]
    ea = ea_parts_ref[0] + ea_parts_ref[1]
    nbr = nbr + jnp.dot(ea, wep_ref[...], preferred_element_type=jnp.float32)
    z = (jnp.dot(h, wr_ref[...], preferred_element_type=jnp.float32)
         + jnp.dot(nbr, wn_ref[...], preferred_element_type=jnp.float32)
         + b_ref[...])
    m = jnp.mean(z, axis=0, keepdims=True)
    d = z - m
    v = jnp.mean(d * d, axis=0, keepdims=True)
    out_ref[...] = jnp.maximum(d * jax.lax.rsqrt(v + 1e-5) * g_ref[...]
                               + be_ref[...], 0.0)


def _head_body(sums_ref, cnt_ref, wf1_ref, bf1_ref, wf2_ref, bf2_ref, out_ref):
    sums = sums_ref[0] + sums_ref[1]
    cnt = cnt_ref[0] + cnt_ref[1]
    csafe = jnp.maximum(cnt[:, 0:1], 1.0)
    hsub = sums / csafe
    rows = lax.broadcasted_iota(jnp.int32, (_G, _TS), 0)
    cols = lax.broadcasted_iota(jnp.int32, (_G, _TS), 1)
    pool = jnp.where(cols // (_TS // _G) == rows, 1.0 / (_TS // _G), 0.0)
    hg = jnp.dot(pool, hsub, preferred_element_type=jnp.float32)
    t = jnp.maximum(jnp.dot(hg, wf1_ref[...], preferred_element_type=jnp.float32)
                    + bf1_ref[...], 0.0)
    out_ref[...] = (jnp.dot(t, wf2_ref[...], preferred_element_type=jnp.float32)
                    + bf2_ref[...])


def _dense(h, parts, ea_parts, wr, wn, wep, b, g, be):
    return pl.pallas_call(
        _dense_body,
        out_shape=jax.ShapeDtypeStruct((_N, _D), jnp.float32),
    )(h, parts, ea_parts, wr, wn, wep,
      b.reshape(1, -1), g.reshape(1, -1), be.reshape(1, -1))


def _make_agg():
    f32 = jnp.float32
    i32 = jnp.int32
    return pl.kernel(
        _agg_body,
        out_type=jax.ShapeDtypeStruct((_NC, _N, _D), f32),
        mesh=_sc_mesh(),
        scratch_types=[
            pltpu.VMEM_SHARED((_N, _D), f32),
            pltpu.VMEM((_EPT,), i32),
            pltpu.VMEM((_CH,), i32), pltpu.VMEM((_CH,), i32),
            pltpu.VMEM((_CH, _D), f32), pltpu.VMEM((_CH, _D), f32),
            pltpu.VMEM((_TAIL,), i32), pltpu.VMEM((_TAIL, _D), f32),
            pltpu.SemaphoreType.DMA, pltpu.SemaphoreType.DMA,
        ])


def kernel(x, edge_index, edge_attr, batch, subgraph_batch, num_subgraphs,
           subgraph_id_batch, W_root0, W_nbr0, We0, b0, g0, be0,
           W_root1, W_nbr1, We1, b1, g1, be1, Wf1, bf1, Wf2, bf2):
    f32 = jnp.float32
    i32 = jnp.int32
    z128 = jnp.zeros((_RPT, _D), f32)
    ones = jnp.ones((_CH, _D), f32)
    ei_flat = edge_index.reshape(-1)
    We0p = jnp.pad(We0, ((0, _D - _DE), (0, 0)))
    We1p = jnp.pad(We1, ((0, _D - _DE), (0, 0)))
    ea_flat = edge_attr.reshape(-1)

    agg = _make_agg()
    parts0 = agg(ei_flat, x, z128)

    ea_k = pl.kernel(
        _ea_body,
        out_type=jax.ShapeDtypeStruct((_NC, _N, _D), f32),
        mesh=_sc_mesh(),
        scratch_types=[
            pltpu.VMEM_SHARED((_N, _D), f32),
            pltpu.VMEM((_CH,), i32), pltpu.VMEM((_CH,), i32),
            pltpu.VMEM((_CH * _DE,), f32), pltpu.VMEM((_CH * _DE,), f32),
            pltpu.VMEM((_CH, _D), f32), pltpu.VMEM((_CH, _D), f32),
            pltpu.VMEM((_TAIL,), i32), pltpu.VMEM((_TAIL * _DE,), f32),
            pltpu.VMEM((_TAIL, _D), f32),
            pltpu.SemaphoreType.DMA, pltpu.SemaphoreType.DMA,
        ])
    ea_parts = ea_k(ei_flat, ea_flat, z128)

    h1 = _dense(x, parts0, ea_parts, W_root0, W_nbr0, We0p, b0, g0, be0)
    parts1 = _make_agg()(ei_flat, h1, z128)
    h2 = _dense(h1, parts1, ea_parts, W_root1, W_nbr1, We1p, b1, g1, be1)

    tmp = jnp.concatenate([jnp.zeros((1,), i32),
                           jnp.cumsum(num_subgraphs, dtype=i32)])
    sid = subgraph_batch + tmp[batch]

    pool = pl.kernel(
        _pool_body,
        out_type=(jax.ShapeDtypeStruct((_NC, _TS, _D), f32),
                  jax.ShapeDtypeStruct((_NC, _TS, _D), f32)),
        mesh=_sc_mesh(),
        scratch_types=[
            pltpu.VMEM_SHARED((_TS, _D), f32),
            pltpu.VMEM_SHARED((_TS, _D), f32),
            pltpu.VMEM((_CH, _D), f32), pltpu.VMEM((_CH,), i32),
            pltpu.VMEM((_NTAIL, _D), f32), pltpu.VMEM((_NTAIL,), i32),
            pltpu.VMEM((_CH, _D), f32),
        ])
    sums, cnts = pool(h2, sid, z128, ones)

    out = pl.pallas_call(
        _head_body,
        out_shape=jax.ShapeDtypeStruct((_G, _TASKS), jnp.float32),
    )(sums, cnts, Wf1, bf1.reshape(1, -1), Wf2, bf2.reshape(1, -1))
    return out
